# baseline (speedup 1.0000x reference)
import functools

import jax
import jax.numpy as jnp
from jax import lax
from jax.experimental import pallas as pl
from jax.experimental.pallas import tpu as pltpu
from jax.experimental.pallas import tpu_sc as plsc

_NUM_EMBEDDINGS = 8192
_EMBEDDING_DIM = 64
_COMMITMENT_COST = 0.25
_TILE = 256


def _vq_body(x_ref, w_ref,
             d_ref, e_ref, idx_ref, loss_ref, perp_ref, wpad_ref,
             loss_acc, counts_acc):
    i = pl.program_id(0)
    nsteps = pl.num_programs(0)

    x = x_ref[...]
    w = w_ref[...]
    x2 = jnp.sum(x * x, axis=1, keepdims=True)
    e2 = jnp.sum(w * w, axis=1)[None, :]
    xw = jax.lax.dot_general(x, w, (((1,), (1,)), ((), ())),
                             preferred_element_type=jnp.float32)
    dist = x2 + e2 - 2.0 * xw
    d_ref[...] = dist

    idx = jnp.argmin(dist, axis=1)
    idx_ref[...] = idx[:, None].astype(jnp.int32)

    col = jax.lax.broadcasted_iota(jnp.int32, dist.shape, 1)
    enc = (col == idx[:, None]).astype(jnp.float32)
    e_ref[...] = enc

    part = jnp.sum(jnp.min(dist, axis=1)).reshape(1, 1)
    cpart = jnp.sum(enc, axis=0, keepdims=True)

    @pl.when(i == 0)
    def _init():
        loss_acc[...] = part
        counts_acc[...] = cpart
        wpad_ref[...] = jnp.concatenate(
            [w, jnp.zeros_like(w)], axis=1)

    @pl.when(i > 0)
    def _accum():
        loss_acc[...] += part
        counts_acc[...] += cpart

    @pl.when(i == nsteps - 1)
    def _finalize():
        n_tokens = nsteps * _TILE
        n_elems = jnp.float32(n_tokens * _EMBEDDING_DIM)
        loss_ref[...] = loss_acc[...] * (_COMMITMENT_COST / n_elems)
        p = counts_acc[...] / jnp.float32(n_tokens)
        perp_ref[...] = jnp.exp(-jnp.sum(p * jnp.log(p + 1e-10))).reshape(1, 1)


def _make_sc_gather(n, k, d):
    info = plsc.get_sparse_core_info()
    nw = info.num_cores * info.num_subcores
    nc = info.num_cores
    b_per_w = n // nw
    mesh = plsc.VectorSubcoreMesh(core_axis_name="c", subcore_axis_name="s")

    @functools.partial(
        pl.kernel, mesh=mesh,
        out_type=jax.ShapeDtypeStruct((n, d), jnp.float32),
        scratch_types=[
            pltpu.VMEM((b_per_w,), jnp.int32),
            pltpu.VMEM((b_per_w, d), jnp.float32),
            pltpu.SemaphoreType.DMA,
        ],
    )
    def _gather(table_hbm, idx_hbm, out_hbm, idx_v, rows_v, sem):
        wid = lax.axis_index("s") * nc + lax.axis_index("c")
        base = wid * b_per_w
        pltpu.sync_copy(idx_hbm.at[pl.ds(base, b_per_w)], idx_v)
        pltpu.async_copy(table_hbm.at[idx_v], rows_v, sem).wait()
        pltpu.sync_copy(rows_v, out_hbm.at[pl.ds(base, b_per_w)])

    return _gather


def kernel(inputs, embedding_weight):
    input_shape = inputs.shape
    flat = inputs.reshape(-1, _EMBEDDING_DIM)
    n = flat.shape[0]
    k = embedding_weight.shape[0]
    nsteps = n // _TILE

    out_shapes = (
        jax.ShapeDtypeStruct((n, k), jnp.float32),
        jax.ShapeDtypeStruct((n, k), jnp.float32),
        jax.ShapeDtypeStruct((n, 1), jnp.int32),
        jax.ShapeDtypeStruct((1, 1), jnp.float32),
        jax.ShapeDtypeStruct((1, 1), jnp.float32),
        jax.ShapeDtypeStruct((k, 2 * _EMBEDDING_DIM), jnp.float32),
    )
    d, e, idx, loss, perp, wpad = pl.pallas_call(
        _vq_body,
        grid=(nsteps,),
        in_specs=[
            pl.BlockSpec((_TILE, _EMBEDDING_DIM), lambda i: (i, 0)),
            pl.BlockSpec((k, _EMBEDDING_DIM), lambda i: (0, 0)),
        ],
        out_specs=[
            pl.BlockSpec((_TILE, k), lambda i: (i, 0)),
            pl.BlockSpec((_TILE, k), lambda i: (i, 0)),
            pl.BlockSpec((_TILE, 1), lambda i: (i, 0)),
            pl.BlockSpec((1, 1), lambda i: (0, 0)),
            pl.BlockSpec((1, 1), lambda i: (0, 0)),
            pl.BlockSpec((k, 2 * _EMBEDDING_DIM), lambda i: (0, 0)),
        ],
        out_shape=out_shapes,
        scratch_shapes=[
            pltpu.VMEM((1, 1), jnp.float32),
            pltpu.VMEM((1, k), jnp.float32),
        ],
    )(flat, embedding_weight)

    q = _make_sc_gather(n, k, 2 * _EMBEDDING_DIM)(
        wpad, idx.reshape(-1))

    vq_loss = loss[0, 0]
    quantized_st = q[:, :_EMBEDDING_DIM].reshape(input_shape)
    perplexity = perp[0, 0]
    return (vq_loss, quantized_st, perplexity, e, d, idx)


# speedup vs baseline: 1.0585x; 1.0585x over previous
import functools

import jax
import jax.numpy as jnp
from jax import lax
from jax.experimental import pallas as pl
from jax.experimental.pallas import tpu as pltpu
from jax.experimental.pallas import tpu_sc as plsc

_NUM_EMBEDDINGS = 8192
_EMBEDDING_DIM = 64
_COMMITMENT_COST = 0.25
_TILE = 256


def _vq_body(x_ref, w_ref,
             d_ref, e_ref, idx_ref, loss_ref, perp_ref, wpad_ref,
             loss_acc, counts_acc, colf_acc):
    i = pl.program_id(0)
    nsteps = pl.num_programs(0)

    x = x_ref[...]
    dcap = wpad_ref.shape[1]

    @pl.when(i == 0)
    def _prologue():
        w0 = w_ref[...]
        kk, dd = w0.shape
        e2col = jax.lax.dot_general(
            w0 * w0, jnp.ones((dd, 1), jnp.float32),
            (((1,), (0,)), ((), ())),
            preferred_element_type=jnp.float32)
        wpad_ref[...] = jnp.concatenate(
            [w0, jnp.ones((kk, 1), jnp.float32), e2col,
             jnp.zeros((kk, dcap - dd - 2), jnp.float32)], axis=1)
        colf_acc[...] = jax.lax.broadcasted_iota(
            jnp.int32, (1, counts_acc.shape[1]), 1).astype(jnp.float32)

    x2 = jax.lax.dot_general(
        x * x, jnp.ones((x.shape[1], 1), jnp.float32),
        (((1,), (0,)), ((), ())),
        preferred_element_type=jnp.float32)
    tt, dd = x.shape
    x_aug = jnp.concatenate(
        [-2.0 * x, x2, jnp.ones((tt, 1), jnp.float32),
         jnp.zeros((tt, dcap - dd - 2), jnp.float32)], axis=1)
    dist = jax.lax.dot_general(
        x_aug, wpad_ref[...], (((1,), (1,)), ((), ())),
        preferred_element_type=jnp.float32)
    d_ref[...] = dist

    rowmin = jnp.min(dist, axis=1, keepdims=True)
    colf = colf_acc[...]
    kf = jnp.float32(dist.shape[1])
    masked_col = jnp.where(dist == rowmin, colf, kf)
    idxf = jnp.min(masked_col, axis=1, keepdims=True)
    idx_ref[...] = idxf.astype(jnp.int32)

    enc = (colf == idxf).astype(jnp.float32)
    e_ref[...] = enc

    part = jnp.sum(rowmin).reshape(1, 1)
    ones_row = jnp.ones((1, enc.shape[0]), jnp.float32)
    cpart = jax.lax.dot_general(ones_row, enc, (((1,), (0,)), ((), ())),
                                preferred_element_type=jnp.float32)

    @pl.when(i == 0)
    def _init():
        loss_acc[...] = part
        counts_acc[...] = cpart

    @pl.when(i > 0)
    def _accum():
        loss_acc[...] += part
        counts_acc[...] += cpart

    @pl.when(i == nsteps - 1)
    def _finalize():
        n_tokens = nsteps * _TILE
        n_elems = jnp.float32(n_tokens * _EMBEDDING_DIM)
        loss_ref[...] = loss_acc[...] * (_COMMITMENT_COST / n_elems)
        p = counts_acc[...] / jnp.float32(n_tokens)
        perp_ref[...] = jnp.exp(-jnp.sum(p * jnp.log(p + 1e-10))).reshape(1, 1)


def _make_sc_gather(n, k, d):
    info = plsc.get_sparse_core_info()
    nw = info.num_cores * info.num_subcores
    nc = info.num_cores
    b_per_w = n // nw
    mesh = plsc.VectorSubcoreMesh(core_axis_name="c", subcore_axis_name="s")

    @functools.partial(
        pl.kernel, mesh=mesh,
        out_type=jax.ShapeDtypeStruct((n, d), jnp.float32),
        scratch_types=[
            pltpu.VMEM((b_per_w,), jnp.int32),
            pltpu.VMEM((b_per_w, d), jnp.float32),
            pltpu.SemaphoreType.DMA,
        ],
    )
    def _gather(table_hbm, idx_hbm, out_hbm, idx_v, rows_v, sem):
        wid = lax.axis_index("s") * nc + lax.axis_index("c")
        base = wid * b_per_w
        pltpu.sync_copy(idx_hbm.at[pl.ds(base, b_per_w)], idx_v)
        pltpu.async_copy(table_hbm.at[idx_v], rows_v, sem).wait()
        pltpu.sync_copy(rows_v, out_hbm.at[pl.ds(base, b_per_w)])

    return _gather


def kernel(inputs, embedding_weight):
    input_shape = inputs.shape
    flat = inputs.reshape(-1, _EMBEDDING_DIM)
    n = flat.shape[0]
    k = embedding_weight.shape[0]
    nsteps = n // _TILE

    out_shapes = (
        jax.ShapeDtypeStruct((n, k), jnp.float32),
        jax.ShapeDtypeStruct((n, k), jnp.float32),
        jax.ShapeDtypeStruct((n, 1), jnp.int32),
        jax.ShapeDtypeStruct((1, 1), jnp.float32),
        jax.ShapeDtypeStruct((1, 1), jnp.float32),
        jax.ShapeDtypeStruct((k, 2 * _EMBEDDING_DIM), jnp.float32),
    )
    d, e, idx, loss, perp, wpad = pl.pallas_call(
        _vq_body,
        grid=(nsteps,),
        in_specs=[
            pl.BlockSpec((_TILE, _EMBEDDING_DIM), lambda i: (i, 0)),
            pl.BlockSpec((k, _EMBEDDING_DIM), lambda i: (0, 0)),
        ],
        out_specs=[
            pl.BlockSpec((_TILE, k), lambda i: (i, 0)),
            pl.BlockSpec((_TILE, k), lambda i: (i, 0)),
            pl.BlockSpec((_TILE, 1), lambda i: (i, 0)),
            pl.BlockSpec((1, 1), lambda i: (0, 0)),
            pl.BlockSpec((1, 1), lambda i: (0, 0)),
            pl.BlockSpec((k, 2 * _EMBEDDING_DIM), lambda i: (0, 0)),
        ],
        out_shape=out_shapes,
        scratch_shapes=[
            pltpu.VMEM((1, 1), jnp.float32),
            pltpu.VMEM((1, k), jnp.float32),
            pltpu.VMEM((1, k), jnp.float32),
        ],
    )(flat, embedding_weight)

    q = _make_sc_gather(n, k, 2 * _EMBEDDING_DIM)(
        wpad, idx.reshape(-1))

    vq_loss = loss[0, 0]
    quantized_st = q[:, :_EMBEDDING_DIM].reshape(input_shape)
    perplexity = perp[0, 0]
    return (vq_loss, quantized_st, perplexity, e, d, idx)
